# baseline (speedup 1.0000x reference)
import jax
import jax.numpy as jnp
from jax.experimental import pallas as pl
from jax.experimental.pallas import tpu as pltpu

_NPOINTS = [256, 128, 64, 32]
_NSAMPLES = [8, 16, 32, 32]
_EPS = 1e-8
_BIG = 3.0e38


def _first_argmin_cols(x, iota, n):
    m = jnp.min(x, axis=1, keepdims=True)
    am = jnp.min(jnp.where(x == m, iota, n), axis=1, keepdims=True)
    return m, am


def _mimic_dists(q_xyz, r_rows):
    sq = jnp.sum(q_xyz * q_xyz, axis=1, keepdims=True)
    sr = (r_rows[0:1, :] * r_rows[0:1, :]
          + r_rows[1:2, :] * r_rows[1:2, :]
          + r_rows[2:3, :] * r_rows[2:3, :])
    g = jax.lax.dot_general(
        q_xyz, r_rows[0:3, :],
        (((1,), (0,)), ((), ())), preferred_element_type=jnp.float32)
    return (sq + sr) - 2.0 * g


def _loa_body(xyz_ref, xyzP_ref, out_ref, d_s):
    x = xyz_ref[0]
    xp = xyzP_ref[0]
    n = x.shape[0]
    d2 = jnp.zeros((n, n), jnp.float32)
    for c in range(3):
        col = x[:, c:c + 1]
        row = xp[c:c + 1, :]
        diff = col - row
        d2 = d2 + diff * diff
    d_s[...] = jnp.sqrt(d2)
    md = _mimic_dists(x, xp)
    def step(_, carry):
        dw, msk = carry
        m = jnp.min(dw, axis=0, keepdims=True)
        sel = dw <= m
        msk = msk + sel.astype(jnp.float32)
        dw = jnp.where(sel, _BIG, dw)
        return dw, msk

    _, msk = jax.lax.fori_loop(
        0, 32, step, (md, jnp.zeros((n, n), jnp.float32)))
    d = d_s[...]
    t = jnp.max(msk * d, axis=0, keepdims=True)
    w = msk * (t - d)
    v = (jax.lax.dot_general(w, x, (((0,), (0,)), ((), ())),
                             preferred_element_type=jnp.float32, precision=jax.lax.Precision.HIGHEST)
         - jax.lax.dot_general(w, jnp.ones((n, 1), jnp.float32),
                               (((0,), (0,)), ((), ())),
                               preferred_element_type=jnp.float32, precision=jax.lax.Precision.HIGHEST) * x)
    nrm = jnp.sqrt(jnp.sum(v * v, axis=1, keepdims=True))
    out_ref[0] = v / (nrm + _EPS)


def _fps_level(planes, o_ref, npoint):
    b, n = planes[0].shape
    iota_n = jax.lax.broadcasted_iota(jnp.int32, (b, n), 1)
    iota_p = jax.lax.broadcasted_iota(jnp.int32, (b, npoint), 1)

    def body(i, st):
        dist, far, sel = st
        oh = (iota_n == far).astype(jnp.float32)
        cs = [jnp.sum(oh * a, axis=1, keepdims=True) for a in planes]
        d = ((planes[0] - cs[0]) ** 2 + (planes[1] - cs[1]) ** 2
             + (planes[2] - cs[2]) ** 2)
        dist = jnp.minimum(dist, d)
        m = jnp.max(dist, axis=1, keepdims=True)
        far = jnp.min(jnp.where(dist == m, iota_n, n), axis=1, keepdims=True)
        sel = tuple(jnp.where(iota_p == i, c, s) for c, s in zip(cs, sel))
        return dist, far, sel

    dist0 = jnp.full((b, n), 1e10, jnp.float32)
    far0 = jnp.zeros((b, 1), jnp.int32)
    sel0 = tuple(jnp.zeros((b, npoint), jnp.float32) for _ in range(6))
    _, _, sel = jax.lax.fori_loop(0, npoint, body, (dist0, far0, sel0))
    for c in range(6):
        o_ref[:, c, :] = sel[c]
    return list(sel)


def _fps_body(xyzT_ref, loaT_ref, o1, o2, o3, o4):
    planes = [xyzT_ref[c] for c in range(3)] + [loaT_ref[c] for c in range(3)]
    for o_ref, npoint in ((o1, _NPOINTS[0]), (o2, _NPOINTS[1]),
                          (o3, _NPOINTS[2]), (o4, _NPOINTS[3])):
        planes = _fps_level(planes, o_ref, npoint)


def _run_module(q_xyz, q_loa, r_xyz, r_loa, r_rows, r_feats, k,
                wri, bri, w0, b0, am_s):
    qn = q_xyz.shape[0]
    n = r_xyz.shape[0]
    cf = 0 if r_feats is None else r_feats.shape[1]
    co = w0.shape[1]
    kq = k * qn
    iota = jax.lax.broadcasted_iota(jnp.int32, (qn, n), 1)
    d2 = _mimic_dists(q_xyz, r_rows)

    def kstep(kk, d2c):
        _, am = _first_argmin_cols(d2c, iota, n)
        am_s[pl.ds(kk * qn, qn), :] = am
        return jnp.where(iota == am, _BIG, d2c)

    jax.lax.fori_loop(0, k, kstep, d2)

    iota_kq = jax.lax.broadcasted_iota(jnp.int32, (kq, n), 1)
    ohall = (iota_kq == am_s[0:kq, :]).astype(jnp.float32)
    dng = (((1,), (0,)), ((), ()))
    gx = jax.lax.dot_general(
        ohall, r_xyz, dng, preferred_element_type=jnp.float32,
        precision=jax.lax.Precision.HIGHEST).reshape(k, qn, 3)
    gl = jax.lax.dot_general(
        ohall, r_loa, dng, preferred_element_type=jnp.float32,
        precision=jax.lax.Precision.HIGHEST).reshape(k, qn, 3)
    gf = None
    if r_feats is not None:
        gf = jax.lax.dot_general(
            ohall, r_feats, dng, preferred_element_type=jnp.float32,
            precision=jax.lax.Precision.HIGHEST)
    rel = gx - q_xyz[None]
    dn = jnp.sqrt(jnp.sum(rel * rel, axis=-1, keepdims=True))
    u = rel / (dn + _EPS)
    c1 = jnp.sum(u * q_loa[None], axis=-1, keepdims=True)
    c2 = jnp.sum(u * gl, axis=-1, keepdims=True)
    c3 = jnp.sum(q_loa[None] * gl, axis=-1, keepdims=True)
    ri = jnp.concatenate([dn, c1, c2, c3], axis=-1).reshape(kq, 4)
    mm = (((1,), (0,)), ((), ()))
    h = jax.nn.relu(jax.lax.dot_general(
        ri, wri, mm, preferred_element_type=jnp.float32, precision=jax.lax.Precision.HIGHEST) + bri)
    z = jax.lax.dot_general(h, w0[0:64, :], mm,
                            preferred_element_type=jnp.float32, precision=jax.lax.Precision.HIGHEST)
    if r_feats is not None:
        z = z + jax.lax.dot_general(gf, w0[64:64 + cf, :], mm,
                                    preferred_element_type=jnp.float32, precision=jax.lax.Precision.HIGHEST)
    z = jax.nn.relu(z + b0)
    return jnp.max(z.reshape(k, qn, co), axis=0)


def _modules_body(xyz_ref, loa_ref, xyzP_ref,
                  nx1_ref, nl1_ref, nx2_ref, nl2_ref,
                  nx3_ref, nl3_ref, nx4_ref, nl4_ref,
                  o1_ref, o2_ref, o3_ref, o4_ref,
                  w1ri, b1ri, w10, b10, w2ri, b2ri, w20, b20,
                  w3ri, b3ri, w30, b30, w4ri, b4ri, w40, b40,
                  w5ri, b5ri, w50, b50,
                  out_ref, am_s):
    xyz = xyz_ref[0]
    loa = loa_ref[0]
    nx = [nx1_ref[0], nx2_ref[0], nx3_ref[0], nx4_ref[0]]
    nl = [nl1_ref[0], nl2_ref[0], nl3_ref[0], nl4_ref[0]]
    rows = [xyzP_ref[0], o1_ref[0], o2_ref[0], o3_ref[0]]
    mp = [(w1ri, b1ri, w10, b10), (w2ri, b2ri, w20, b20),
          (w3ri, b3ri, w30, b30), (w4ri, b4ri, w40, b40)]

    f = None
    r_xyz, r_loa = xyz, loa
    for m in range(4):
        wri, bri, w0, b0 = mp[m]
        f = _run_module(nx[m], nl[m], r_xyz, r_loa, rows[m], f,
                        _NSAMPLES[m],
                        wri[...], bri[...], w0[...], b0[...], am_s)
        r_xyz, r_loa = nx[m], nl[m]

    r_xyz, r_loa, r_feats = nx[3], nl[3], f
    q_xyz = jnp.mean(r_xyz, axis=0, keepdims=True)
    v5 = jnp.sum(r_loa, axis=0, keepdims=True)
    q_loa = v5 / (jnp.sqrt(jnp.sum(v5 * v5, axis=-1, keepdims=True)) + _EPS)
    rel = r_xyz - q_xyz
    dn = jnp.sqrt(jnp.sum(rel * rel, axis=-1, keepdims=True))
    u = rel / (dn + _EPS)
    c1 = jnp.sum(u * q_loa, axis=-1, keepdims=True)
    c2 = jnp.sum(u * r_loa, axis=-1, keepdims=True)
    c3 = jnp.sum(q_loa * r_loa, axis=-1, keepdims=True)
    ri = jnp.concatenate([dn, c1, c2, c3], axis=-1)
    mm = (((1,), (0,)), ((), ()))
    h = jax.nn.relu(jax.lax.dot_general(
        ri, w5ri[...], mm, preferred_element_type=jnp.float32, precision=jax.lax.Precision.HIGHEST) + b5ri[...])
    z = (jax.lax.dot_general(h, w50[0:64, :], mm,
                             preferred_element_type=jnp.float32, precision=jax.lax.Precision.HIGHEST)
         + jax.lax.dot_general(r_feats, w50[64:320, :], mm,
                               preferred_element_type=jnp.float32, precision=jax.lax.Precision.HIGHEST))
    z = jax.nn.relu(z + b50[...])
    out_ref[0] = jnp.max(z, axis=0, keepdims=True)


def _head_body(f5_ref, w1, b1, g1, bb1, w2, b2, g2, bb2, w3, b3, out_ref):
    mm = (((1,), (0,)), ((), ()))
    x = f5_ref[...]
    x = jax.nn.relu(g1[...] * (jax.lax.dot_general(
        x, w1[...], mm, preferred_element_type=jnp.float32, precision=jax.lax.Precision.HIGHEST) + b1[...])
        + bb1[...])
    x = jax.nn.relu(g2[...] * (jax.lax.dot_general(
        x, w2[...], mm, preferred_element_type=jnp.float32, precision=jax.lax.Precision.HIGHEST) + b2[...])
        + bb2[...])
    x = jax.lax.dot_general(
        x, w3[...], mm, preferred_element_type=jnp.float32, precision=jax.lax.Precision.HIGHEST) + b3[...]
    m = jnp.max(x, axis=-1, keepdims=True)
    lse = jnp.log(jnp.sum(jnp.exp(x - m), axis=-1, keepdims=True))
    out_ref[...] = x - m - lse


def _full_spec(shape):
    nd = len(shape)
    return pl.BlockSpec(shape, lambda *_a, _n=nd: (0,) * _n)


def kernel(xyz, params):
    b, n, _ = xyz.shape
    f32 = jnp.float32
    xyzT = jnp.transpose(xyz, (2, 0, 1))
    xyzP = jnp.transpose(xyz, (0, 2, 1))

    loa = pl.pallas_call(
        _loa_body,
        grid=(b,),
        in_specs=[pl.BlockSpec((1, n, 3), lambda i: (i, 0, 0)),
                  pl.BlockSpec((1, 3, n), lambda i: (i, 0, 0))],
        out_specs=pl.BlockSpec((1, n, 3), lambda i: (i, 0, 0)),
        out_shape=jax.ShapeDtypeStruct((b, n, 3), f32),
        scratch_shapes=[pltpu.VMEM((n, n), f32)],
        compiler_params=pltpu.CompilerParams(
            dimension_semantics=("parallel",)),
    )(xyz, xyzP)

    loaT = jnp.transpose(loa, (2, 0, 1))

    fps_outs = pl.pallas_call(
        _fps_body,
        in_specs=[_full_spec((3, b, n)), _full_spec((3, b, n))],
        out_specs=[_full_spec((b, 6, p)) for p in _NPOINTS],
        out_shape=[jax.ShapeDtypeStruct((b, 6, p), f32) for p in _NPOINTS],
    )(xyzT, loaT)

    nx = [jnp.transpose(o[:, 0:3, :], (0, 2, 1)) for o in fps_outs]
    nl = [jnp.transpose(o[:, 3:6, :], (0, 2, 1)) for o in fps_outs]

    p = params
    mparams = []
    for m in range(1, 6):
        mparams += [p['m%d_Wri' % m], p['m%d_bri' % m].reshape(1, -1),
                    p['m%d_W0' % m], p['m%d_b0' % m].reshape(1, -1)]

    in_specs = [pl.BlockSpec((1, n, 3), lambda i: (i, 0, 0)),
                pl.BlockSpec((1, n, 3), lambda i: (i, 0, 0)),
                pl.BlockSpec((1, 3, n), lambda i: (i, 0, 0))]
    for pts in _NPOINTS:
        in_specs += [pl.BlockSpec((1, pts, 3), lambda i: (i, 0, 0))] * 2
    for pts in _NPOINTS:
        in_specs.append(pl.BlockSpec((1, 6, pts), lambda i: (i, 0, 0)))
    for w in mparams:
        in_specs.append(_full_spec(w.shape))

    args = [xyz, loa, xyzP]
    for m in range(4):
        args += [nx[m], nl[m]]
    args += list(fps_outs)
    args += mparams

    f5 = pl.pallas_call(
        _modules_body,
        grid=(b,),
        in_specs=in_specs,
        out_specs=pl.BlockSpec((1, 1, 512), lambda i: (i, 0, 0)),
        out_shape=jax.ShapeDtypeStruct((b, 1, 512), f32),
        scratch_shapes=[pltpu.VMEM((2048, 1), jnp.int32)],
        compiler_params=pltpu.CompilerParams(
            dimension_semantics=("parallel",)),
    )(*args)

    hp = [p['fc1_W'], p['fc1_b'].reshape(1, -1),
          p['bn1_g'].reshape(1, -1), p['bn1_b'].reshape(1, -1),
          p['fc2_W'], p['fc2_b'].reshape(1, -1),
          p['bn2_g'].reshape(1, -1), p['bn2_b'].reshape(1, -1),
          p['fc3_W'], p['fc3_b'].reshape(1, -1)]
    logp = pl.pallas_call(
        _head_body,
        in_specs=[_full_spec((b, 512))] + [_full_spec(w.shape) for w in hp],
        out_specs=_full_spec((b, 40)),
        out_shape=jax.ShapeDtypeStruct((b, 40), f32),
    )(f5.reshape(b, 512), *hp)

    return logp, f5


# speedup vs baseline: 1.0612x; 1.0612x over previous
import functools

import jax
import jax.numpy as jnp
from jax import lax
from jax.experimental import pallas as pl
from jax.experimental.pallas import tpu as pltpu
from jax.experimental.pallas import tpu_sc as plsc

_NPOINTS = [256, 128, 64, 32]
_NSAMPLES = [8, 16, 32, 32]
_EPS = 1e-8
_BIG = 3.0e38


def _first_argmin_cols(x, iota, n):
    m = jnp.min(x, axis=1, keepdims=True)
    am = jnp.min(jnp.where(x == m, iota, n), axis=1, keepdims=True)
    return m, am


def _mimic_dists(q_xyz, r_rows):
    sq = jnp.sum(q_xyz * q_xyz, axis=1, keepdims=True)
    sr = (r_rows[0:1, :] * r_rows[0:1, :]
          + r_rows[1:2, :] * r_rows[1:2, :]
          + r_rows[2:3, :] * r_rows[2:3, :])
    g = jax.lax.dot_general(
        q_xyz, r_rows[0:3, :],
        (((1,), (0,)), ((), ())), preferred_element_type=jnp.float32)
    return (sq + sr) - 2.0 * g


def _loa_body(xyz_ref, xyzP_ref, out_ref, d_s):
    x = xyz_ref[0]
    xp = xyzP_ref[0]
    n = x.shape[0]
    d2 = jnp.zeros((n, n), jnp.float32)
    for c in range(3):
        col = x[:, c:c + 1]
        row = xp[c:c + 1, :]
        diff = col - row
        d2 = d2 + diff * diff
    d_s[...] = jnp.sqrt(d2)
    md = _mimic_dists(x, xp)
    def step(_, carry):
        dw, msk = carry
        m = jnp.min(dw, axis=1, keepdims=True)
        sel = dw <= m
        msk = msk + sel.astype(jnp.float32)
        dw = jnp.where(sel, _BIG, dw)
        return dw, msk

    _, msk = jax.lax.fori_loop(
        0, 32, step, (md, jnp.zeros((n, n), jnp.float32)))
    d = d_s[...]
    t = jnp.max(msk * d, axis=1, keepdims=True)
    w = msk * (t - d)
    v = (jax.lax.dot_general(w, x, (((1,), (0,)), ((), ())),
                             preferred_element_type=jnp.float32, precision=jax.lax.Precision.HIGHEST)
         - jnp.sum(w, axis=1, keepdims=True) * x)
    nrm = jnp.sqrt(jnp.sum(v * v, axis=1, keepdims=True))
    out_ref[0] = v / (nrm + _EPS)


def _sc_level(srcs, dsts, dist, n, npoint):
    nsl = n // 16
    lane = lax.broadcasted_iota(jnp.int32, (16,), 0)

    def init_s(s, carry):
        dist[pl.ds(s * 16, 16)] = jnp.full((16,), 1e10, jnp.float32)
        return carry

    lax.fori_loop(0, nsl, init_s, 0)

    def body(i, far):
        cs = [srcs[c][pl.ds(far, 16)][0] for c in range(6)]
        blk = (i // 16) * 16
        sel_st = lane == (i - blk)
        for c in range(6):
            old = dsts[c][pl.ds(blk, 16)]
            dsts[c][pl.ds(blk, 16)] = jnp.where(
                sel_st, lax.broadcast(cs[c], (16,)), old)
        csv = [lax.broadcast(v, (16,)) for v in cs[:3]]

        def scan_s(s, carry):
            mv, ms = carry
            off = s * 16
            dx = srcs[0][pl.ds(off, 16)] - csv[0]
            dy = srcs[1][pl.ds(off, 16)] - csv[1]
            dz = srcs[2][pl.ds(off, 16)] - csv[2]
            d = dx * dx + dy * dy + dz * dz
            dn = jnp.minimum(dist[pl.ds(off, 16)], d)
            dist[pl.ds(off, 16)] = dn
            upd = dn > mv
            mv = jnp.where(upd, dn, mv)
            ms = jnp.where(upd, lax.broadcast(s, (16,)), ms)
            return mv, ms

        mv, ms = lax.fori_loop(
            0, nsl, scan_s, (jnp.full((16,), -1.0, jnp.float32),
                             jnp.zeros((16,), jnp.int32)))
        best = jnp.array(-1.0, jnp.float32)
        bidx = jnp.array(0, jnp.int32)
        for j in range(16):
            v = mv[j]
            cnd = ms[j] * 16 + j
            better = (v > best) | ((v == best) & (cnd < bidx))
            best = jnp.where(better, v, best)
            bidx = jnp.where(better, cnd, bidx)
        return bidx

    lax.fori_loop(0, npoint, body, jnp.array(0, jnp.int32))


def _sc_fps(xyzT, loaT, b):
    f32 = jnp.float32
    mesh = plsc.VectorSubcoreMesh(core_axis_name="c", subcore_axis_name="s")
    out_type = [jax.ShapeDtypeStruct((b, 6, p), f32) for p in _NPOINTS]
    scratch = ([pltpu.VMEM((1024,), f32)] * 6
               + [pltpu.VMEM((256,), f32)] * 6
               + [pltpu.VMEM((128,), f32)] * 6
               + [pltpu.VMEM((64,), f32)] * 6
               + [pltpu.VMEM((32,), f32)] * 6
               + [pltpu.VMEM((1024,), f32), pltpu.VMEM((32,), f32)])

    @functools.partial(pl.kernel, mesh=mesh, out_type=out_type,
                       scratch_types=scratch)
    def k(xyzT_hbm, loaT_hbm, o1, o2, o3, o4, *bufs):
        pin = list(bufs[0:6])
        lv = [list(bufs[6:12]), list(bufs[12:18]),
              list(bufs[18:24]), list(bufs[24:30])]
        dist = bufs[30]
        wid = lax.axis_index("s") * 2 + lax.axis_index("c")

        def sample(j, carry):
            bb = wid * 2 + j
            for c in range(3):
                pltpu.sync_copy(xyzT_hbm.at[c, bb], pin[c])
                pltpu.sync_copy(loaT_hbm.at[c, bb], pin[c + 3])
            for srcs, dsts, o_hbm, n, p in ((pin, lv[0], o1, 1024, 256),
                                            (lv[0], lv[1], o2, 256, 128),
                                            (lv[1], lv[2], o3, 128, 64),
                                            (lv[2], lv[3], o4, 64, 32)):
                _sc_level(srcs, dsts, dist, n, p)
                for c in range(6):
                    pltpu.sync_copy(dsts[c], o_hbm.at[bb, c])
            return carry

        lax.fori_loop(0, 2, sample, 0)

    return k(xyzT, loaT)


def _fps_level(planes, o_ref, npoint):
    b, n = planes[0].shape
    iota_n = jax.lax.broadcasted_iota(jnp.int32, (b, n), 1)
    iota_p = jax.lax.broadcasted_iota(jnp.int32, (b, npoint), 1)

    def body(i, st):
        dist, far, sel = st
        oh = (iota_n == far).astype(jnp.float32)
        cs = [jnp.sum(oh * a, axis=1, keepdims=True) for a in planes]
        d = ((planes[0] - cs[0]) ** 2 + (planes[1] - cs[1]) ** 2
             + (planes[2] - cs[2]) ** 2)
        dist = jnp.minimum(dist, d)
        m = jnp.max(dist, axis=1, keepdims=True)
        far = jnp.min(jnp.where(dist == m, iota_n, n), axis=1, keepdims=True)
        sel = tuple(jnp.where(iota_p == i, c, s) for c, s in zip(cs, sel))
        return dist, far, sel

    dist0 = jnp.full((b, n), 1e10, jnp.float32)
    far0 = jnp.zeros((b, 1), jnp.int32)
    sel0 = tuple(jnp.zeros((b, npoint), jnp.float32) for _ in range(6))
    _, _, sel = jax.lax.fori_loop(0, npoint, body, (dist0, far0, sel0))
    for c in range(6):
        o_ref[:, c, :] = sel[c]
    return list(sel)


def _fps_body(xyzT_ref, loaT_ref, o1, o2, o3, o4):
    planes = [xyzT_ref[c] for c in range(3)] + [loaT_ref[c] for c in range(3)]
    for o_ref, npoint in ((o1, _NPOINTS[0]), (o2, _NPOINTS[1]),
                          (o3, _NPOINTS[2]), (o4, _NPOINTS[3])):
        planes = _fps_level(planes, o_ref, npoint)


def _run_module(q_xyz, q_loa, r_xyz, r_loa, r_rows, r_feats, k,
                wri, bri, w0, b0, am_s):
    qn = q_xyz.shape[0]
    n = r_xyz.shape[0]
    cf = 0 if r_feats is None else r_feats.shape[1]
    co = w0.shape[1]
    kq = k * qn
    iota = jax.lax.broadcasted_iota(jnp.int32, (qn, n), 1)
    d2 = _mimic_dists(q_xyz, r_rows)

    def kstep(kk, d2c):
        _, am = _first_argmin_cols(d2c, iota, n)
        am_s[pl.ds(kk * qn, qn), :] = am
        return jnp.where(iota == am, _BIG, d2c)

    jax.lax.fori_loop(0, k, kstep, d2)

    iota_kq = jax.lax.broadcasted_iota(jnp.int32, (kq, n), 1)
    ohall = (iota_kq == am_s[0:kq, :]).astype(jnp.float32)
    dng = (((1,), (0,)), ((), ()))
    gx = jax.lax.dot_general(
        ohall, r_xyz, dng, preferred_element_type=jnp.float32,
        precision=jax.lax.Precision.HIGHEST).reshape(k, qn, 3)
    gl = jax.lax.dot_general(
        ohall, r_loa, dng, preferred_element_type=jnp.float32,
        precision=jax.lax.Precision.HIGHEST).reshape(k, qn, 3)
    gf = None
    if r_feats is not None:
        gf = jax.lax.dot_general(
            ohall, r_feats, dng, preferred_element_type=jnp.float32,
            precision=jax.lax.Precision.HIGHEST)
    rel = gx - q_xyz[None]
    dn = jnp.sqrt(jnp.sum(rel * rel, axis=-1, keepdims=True))
    u = rel / (dn + _EPS)
    c1 = jnp.sum(u * q_loa[None], axis=-1, keepdims=True)
    c2 = jnp.sum(u * gl, axis=-1, keepdims=True)
    c3 = jnp.sum(q_loa[None] * gl, axis=-1, keepdims=True)
    ri = jnp.concatenate([dn, c1, c2, c3], axis=-1).reshape(kq, 4)
    mm = (((1,), (0,)), ((), ()))
    h = jax.nn.relu(jax.lax.dot_general(
        ri, wri, mm, preferred_element_type=jnp.float32, precision=jax.lax.Precision.HIGHEST) + bri)
    z = jax.lax.dot_general(h, w0[0:64, :], mm,
                            preferred_element_type=jnp.float32, precision=jax.lax.Precision.HIGHEST)
    if r_feats is not None:
        z = z + jax.lax.dot_general(gf, w0[64:64 + cf, :], mm,
                                    preferred_element_type=jnp.float32, precision=jax.lax.Precision.HIGHEST)
    z = jax.nn.relu(z + b0)
    return jnp.max(z.reshape(k, qn, co), axis=0)


def _modules_body(xyz_ref, loa_ref, xyzP_ref,
                  nx1_ref, nl1_ref, nx2_ref, nl2_ref,
                  nx3_ref, nl3_ref, nx4_ref, nl4_ref,
                  o1_ref, o2_ref, o3_ref, o4_ref,
                  w1ri, b1ri, w10, b10, w2ri, b2ri, w20, b20,
                  w3ri, b3ri, w30, b30, w4ri, b4ri, w40, b40,
                  w5ri, b5ri, w50, b50,
                  out_ref, am_s):
    xyz = xyz_ref[0]
    loa = loa_ref[0]
    nx = [nx1_ref[0], nx2_ref[0], nx3_ref[0], nx4_ref[0]]
    nl = [nl1_ref[0], nl2_ref[0], nl3_ref[0], nl4_ref[0]]
    rows = [xyzP_ref[0], o1_ref[0], o2_ref[0], o3_ref[0]]
    mp = [(w1ri, b1ri, w10, b10), (w2ri, b2ri, w20, b20),
          (w3ri, b3ri, w30, b30), (w4ri, b4ri, w40, b40)]

    f = None
    r_xyz, r_loa = xyz, loa
    for m in range(4):
        wri, bri, w0, b0 = mp[m]
        f = _run_module(nx[m], nl[m], r_xyz, r_loa, rows[m], f,
                        _NSAMPLES[m],
                        wri[...], bri[...], w0[...], b0[...], am_s)
        r_xyz, r_loa = nx[m], nl[m]

    r_xyz, r_loa, r_feats = nx[3], nl[3], f
    q_xyz = jnp.mean(r_xyz, axis=0, keepdims=True)
    v5 = jnp.sum(r_loa, axis=0, keepdims=True)
    q_loa = v5 / (jnp.sqrt(jnp.sum(v5 * v5, axis=-1, keepdims=True)) + _EPS)
    rel = r_xyz - q_xyz
    dn = jnp.sqrt(jnp.sum(rel * rel, axis=-1, keepdims=True))
    u = rel / (dn + _EPS)
    c1 = jnp.sum(u * q_loa, axis=-1, keepdims=True)
    c2 = jnp.sum(u * r_loa, axis=-1, keepdims=True)
    c3 = jnp.sum(q_loa * r_loa, axis=-1, keepdims=True)
    ri = jnp.concatenate([dn, c1, c2, c3], axis=-1)
    mm = (((1,), (0,)), ((), ()))
    h = jax.nn.relu(jax.lax.dot_general(
        ri, w5ri[...], mm, preferred_element_type=jnp.float32, precision=jax.lax.Precision.HIGHEST) + b5ri[...])
    z = (jax.lax.dot_general(h, w50[0:64, :], mm,
                             preferred_element_type=jnp.float32, precision=jax.lax.Precision.HIGHEST)
         + jax.lax.dot_general(r_feats, w50[64:320, :], mm,
                               preferred_element_type=jnp.float32, precision=jax.lax.Precision.HIGHEST))
    z = jax.nn.relu(z + b50[...])
    out_ref[0] = jnp.max(z, axis=0, keepdims=True)


def _head_body(f5_ref, w1, b1, g1, bb1, w2, b2, g2, bb2, w3, b3, out_ref):
    mm = (((1,), (0,)), ((), ()))
    x = f5_ref[...]
    x = jax.nn.relu(g1[...] * (jax.lax.dot_general(
        x, w1[...], mm, preferred_element_type=jnp.float32, precision=jax.lax.Precision.HIGHEST) + b1[...])
        + bb1[...])
    x = jax.nn.relu(g2[...] * (jax.lax.dot_general(
        x, w2[...], mm, preferred_element_type=jnp.float32, precision=jax.lax.Precision.HIGHEST) + b2[...])
        + bb2[...])
    x = jax.lax.dot_general(
        x, w3[...], mm, preferred_element_type=jnp.float32, precision=jax.lax.Precision.HIGHEST) + b3[...]
    m = jnp.max(x, axis=-1, keepdims=True)
    lse = jnp.log(jnp.sum(jnp.exp(x - m), axis=-1, keepdims=True))
    out_ref[...] = x - m - lse


def _full_spec(shape):
    nd = len(shape)
    return pl.BlockSpec(shape, lambda *_a, _n=nd: (0,) * _n)


def kernel(xyz, params):
    b, n, _ = xyz.shape
    f32 = jnp.float32
    xyzT = jnp.transpose(xyz, (2, 0, 1))
    xyzP = jnp.transpose(xyz, (0, 2, 1))

    loa = pl.pallas_call(
        _loa_body,
        grid=(b,),
        in_specs=[pl.BlockSpec((1, n, 3), lambda i: (i, 0, 0)),
                  pl.BlockSpec((1, 3, n), lambda i: (i, 0, 0))],
        out_specs=pl.BlockSpec((1, n, 3), lambda i: (i, 0, 0)),
        out_shape=jax.ShapeDtypeStruct((b, n, 3), f32),
        scratch_shapes=[pltpu.VMEM((n, n), f32)],
        compiler_params=pltpu.CompilerParams(
            dimension_semantics=("parallel",)),
    )(xyz, xyzP)

    loaT = jnp.transpose(loa, (2, 0, 1))

    fps_outs = _sc_fps(xyzT, loaT, b)

    nx = [jnp.transpose(o[:, 0:3, :], (0, 2, 1)) for o in fps_outs]
    nl = [jnp.transpose(o[:, 3:6, :], (0, 2, 1)) for o in fps_outs]

    p = params
    mparams = []
    for m in range(1, 6):
        mparams += [p['m%d_Wri' % m], p['m%d_bri' % m].reshape(1, -1),
                    p['m%d_W0' % m], p['m%d_b0' % m].reshape(1, -1)]

    in_specs = [pl.BlockSpec((1, n, 3), lambda i: (i, 0, 0)),
                pl.BlockSpec((1, n, 3), lambda i: (i, 0, 0)),
                pl.BlockSpec((1, 3, n), lambda i: (i, 0, 0))]
    for pts in _NPOINTS:
        in_specs += [pl.BlockSpec((1, pts, 3), lambda i: (i, 0, 0))] * 2
    for pts in _NPOINTS:
        in_specs.append(pl.BlockSpec((1, 6, pts), lambda i: (i, 0, 0)))
    for w in mparams:
        in_specs.append(_full_spec(w.shape))

    args = [xyz, loa, xyzP]
    for m in range(4):
        args += [nx[m], nl[m]]
    args += list(fps_outs)
    args += mparams

    f5 = pl.pallas_call(
        _modules_body,
        grid=(b,),
        in_specs=in_specs,
        out_specs=pl.BlockSpec((1, 1, 512), lambda i: (i, 0, 0)),
        out_shape=jax.ShapeDtypeStruct((b, 1, 512), f32),
        scratch_shapes=[pltpu.VMEM((2048, 1), jnp.int32)],
        compiler_params=pltpu.CompilerParams(
            dimension_semantics=("parallel",)),
    )(*args)

    hp = [p['fc1_W'], p['fc1_b'].reshape(1, -1),
          p['bn1_g'].reshape(1, -1), p['bn1_b'].reshape(1, -1),
          p['fc2_W'], p['fc2_b'].reshape(1, -1),
          p['bn2_g'].reshape(1, -1), p['bn2_b'].reshape(1, -1),
          p['fc3_W'], p['fc3_b'].reshape(1, -1)]
    logp = pl.pallas_call(
        _head_body,
        in_specs=[_full_spec((b, 512))] + [_full_spec(w.shape) for w in hp],
        out_specs=_full_spec((b, 40)),
        out_shape=jax.ShapeDtypeStruct((b, 40), f32),
    )(f5.reshape(b, 512), *hp)

    return logp, f5


# speedup vs baseline: 1.2603x; 1.1876x over previous
import functools

import jax
import jax.numpy as jnp
from jax import lax
from jax.experimental import pallas as pl
from jax.experimental.pallas import tpu as pltpu
from jax.experimental.pallas import tpu_sc as plsc

_NPOINTS = [256, 128, 64, 32]
_NSAMPLES = [8, 16, 32, 32]
_EPS = 1e-8
_BIG = 3.0e38


def _first_argmin_cols(x, iota, n):
    m = jnp.min(x, axis=1, keepdims=True)
    am = jnp.min(jnp.where(x == m, iota, n), axis=1, keepdims=True)
    return m, am


def _mimic_dists(q_xyz, r_rows):
    sq = jnp.sum(q_xyz * q_xyz, axis=1, keepdims=True)
    sr = (r_rows[0:1, :] * r_rows[0:1, :]
          + r_rows[1:2, :] * r_rows[1:2, :]
          + r_rows[2:3, :] * r_rows[2:3, :])
    g = jax.lax.dot_general(
        q_xyz, r_rows[0:3, :],
        (((1,), (0,)), ((), ())), preferred_element_type=jnp.float32)
    return (sq + sr) - 2.0 * g


def _loa_body(xyz_ref, xyzP_ref, out_ref, d_s):
    x = xyz_ref[0]
    xp = xyzP_ref[0]
    n = x.shape[0]
    d2 = jnp.zeros((n, n), jnp.float32)
    for c in range(3):
        col = x[:, c:c + 1]
        row = xp[c:c + 1, :]
        diff = col - row
        d2 = d2 + diff * diff
    d_s[...] = jnp.sqrt(d2)
    md = _mimic_dists(x, xp)
    def step(_, carry):
        dw, msk = carry
        m = jnp.min(dw, axis=1, keepdims=True)
        sel = dw <= m
        msk = msk + sel.astype(jnp.float32)
        dw = jnp.where(sel, _BIG, dw)
        return dw, msk

    _, msk = jax.lax.fori_loop(
        0, 32, step, (md, jnp.zeros((n, n), jnp.float32)))
    d = d_s[...]
    t = jnp.max(msk * d, axis=1, keepdims=True)
    w = msk * (t - d)
    v = (jax.lax.dot_general(w, x, (((1,), (0,)), ((), ())),
                             preferred_element_type=jnp.float32, precision=jax.lax.Precision.HIGHEST)
         - jnp.sum(w, axis=1, keepdims=True) * x)
    nrm = jnp.sqrt(jnp.sum(v * v, axis=1, keepdims=True))
    out_ref[0] = v / (nrm + _EPS)


def _sc_level(srcs, dsts, dist, n, npoint):
    nsl = n // 16
    lane = lax.broadcasted_iota(jnp.int32, (16,), 0)

    def init_s(s, carry):
        dist[pl.ds(s * 16, 16)] = jnp.full((16,), 1e10, jnp.float32)
        return carry

    lax.fori_loop(0, nsl, init_s, 0)

    def body(i, far):
        cs = [srcs[c][pl.ds(far, 16)][0] for c in range(6)]
        blk = (i // 16) * 16
        sel_st = lane == (i - blk)
        for c in range(6):
            old = dsts[c][pl.ds(blk, 16)]
            dsts[c][pl.ds(blk, 16)] = jnp.where(
                sel_st, lax.broadcast(cs[c], (16,)), old)
        csv = [lax.broadcast(v, (16,)) for v in cs[:3]]

        def scan_s(s, carry):
            mv, ms = carry
            off = s * 16
            dx = srcs[0][pl.ds(off, 16)] - csv[0]
            dy = srcs[1][pl.ds(off, 16)] - csv[1]
            dz = srcs[2][pl.ds(off, 16)] - csv[2]
            d = dx * dx + dy * dy + dz * dz
            dn = jnp.minimum(dist[pl.ds(off, 16)], d)
            dist[pl.ds(off, 16)] = dn
            upd = dn > mv
            mv = jnp.where(upd, dn, mv)
            ms = jnp.where(upd, lax.broadcast(s, (16,)), ms)
            return mv, ms

        mv, ms = lax.fori_loop(
            0, nsl, scan_s, (jnp.full((16,), -1.0, jnp.float32),
                             jnp.zeros((16,), jnp.int32)))
        best = jnp.array(-1.0, jnp.float32)
        bidx = jnp.array(0, jnp.int32)
        for j in range(16):
            v = mv[j]
            cnd = ms[j] * 16 + j
            better = (v > best) | ((v == best) & (cnd < bidx))
            best = jnp.where(better, v, best)
            bidx = jnp.where(better, cnd, bidx)
        return bidx

    lax.fori_loop(0, npoint, body, jnp.array(0, jnp.int32))


def _sc_fps(xyzT, loaT, b):
    f32 = jnp.float32
    mesh = plsc.VectorSubcoreMesh(core_axis_name="c", subcore_axis_name="s")
    out_type = [jax.ShapeDtypeStruct((b, 6, p), f32) for p in _NPOINTS]
    scratch = ([pltpu.VMEM((1024,), f32)] * 6
               + [pltpu.VMEM((256,), f32)] * 6
               + [pltpu.VMEM((128,), f32)] * 6
               + [pltpu.VMEM((64,), f32)] * 6
               + [pltpu.VMEM((32,), f32)] * 6
               + [pltpu.VMEM((1024,), f32), pltpu.VMEM((32,), f32)])

    @functools.partial(pl.kernel, mesh=mesh, out_type=out_type,
                       scratch_types=scratch)
    def k(xyzT_hbm, loaT_hbm, o1, o2, o3, o4, *bufs):
        pin = list(bufs[0:6])
        lv = [list(bufs[6:12]), list(bufs[12:18]),
              list(bufs[18:24]), list(bufs[24:30])]
        dist = bufs[30]
        wid = lax.axis_index("s") * 2 + lax.axis_index("c")

        def sample(j, carry):
            bb = wid * 2 + j
            for c in range(3):
                pltpu.sync_copy(xyzT_hbm.at[c, bb], pin[c])
                pltpu.sync_copy(loaT_hbm.at[c, bb], pin[c + 3])
            for srcs, dsts, o_hbm, n, p in ((pin, lv[0], o1, 1024, 256),
                                            (lv[0], lv[1], o2, 256, 128),
                                            (lv[1], lv[2], o3, 128, 64),
                                            (lv[2], lv[3], o4, 64, 32)):
                _sc_level(srcs, dsts, dist, n, p)
                for c in range(6):
                    pltpu.sync_copy(dsts[c], o_hbm.at[bb, c])
            return carry

        lax.fori_loop(0, 2, sample, 0)

    return k(xyzT, loaT)


def _fps_level(planes, o_ref, npoint):
    b, n = planes[0].shape
    iota_n = jax.lax.broadcasted_iota(jnp.int32, (b, n), 1)
    iota_p = jax.lax.broadcasted_iota(jnp.int32, (b, npoint), 1)

    def body(i, st):
        dist, far, sel = st
        oh = (iota_n == far).astype(jnp.float32)
        cs = [jnp.sum(oh * a, axis=1, keepdims=True) for a in planes]
        d = ((planes[0] - cs[0]) ** 2 + (planes[1] - cs[1]) ** 2
             + (planes[2] - cs[2]) ** 2)
        dist = jnp.minimum(dist, d)
        m = jnp.max(dist, axis=1, keepdims=True)
        far = jnp.min(jnp.where(dist == m, iota_n, n), axis=1, keepdims=True)
        sel = tuple(jnp.where(iota_p == i, c, s) for c, s in zip(cs, sel))
        return dist, far, sel

    dist0 = jnp.full((b, n), 1e10, jnp.float32)
    far0 = jnp.zeros((b, 1), jnp.int32)
    sel0 = tuple(jnp.zeros((b, npoint), jnp.float32) for _ in range(6))
    _, _, sel = jax.lax.fori_loop(0, npoint, body, (dist0, far0, sel0))
    for c in range(6):
        o_ref[:, c, :] = sel[c]
    return list(sel)


def _fps_body(xyzT_ref, loaT_ref, o1, o2, o3, o4):
    planes = [xyzT_ref[c] for c in range(3)] + [loaT_ref[c] for c in range(3)]
    for o_ref, npoint in ((o1, _NPOINTS[0]), (o2, _NPOINTS[1]),
                          (o3, _NPOINTS[2]), (o4, _NPOINTS[3])):
        planes = _fps_level(planes, o_ref, npoint)


def _make_knn_body(chunk, q, n, k):
    def body(q_ref, rows_ref, out_ref):
        d2s = [_mimic_dists(q_ref[s], rows_ref[s]) for s in range(chunk)]
        d2 = jnp.concatenate(d2s, axis=0) if chunk > 1 else d2s[0]
        iota = jax.lax.broadcasted_iota(jnp.int32, (chunk * q, n), 1)

        def kstep(kk, d2c):
            _, am = _first_argmin_cols(d2c, iota, n)
            for s in range(chunk):
                out_ref[s, pl.ds(kk * q, q), 0:1] = am[s * q:(s + 1) * q]
            return jnp.where(iota == am, _BIG, d2c)

        jax.lax.fori_loop(0, k, kstep, d2)

    return body


def _run_module(q_xyz, q_loa, r_xyz, r_loa, am_col, r_feats, k,
                wri, bri, w0, b0):
    qn = q_xyz.shape[0]
    n = r_xyz.shape[0]
    cf = 0 if r_feats is None else r_feats.shape[1]
    co = w0.shape[1]
    kq = k * qn
    iota_kq = jax.lax.broadcasted_iota(jnp.int32, (kq, n), 1)
    ohall = (iota_kq == am_col).astype(jnp.float32)
    dng = (((1,), (0,)), ((), ()))
    gx = jax.lax.dot_general(
        ohall, r_xyz, dng, preferred_element_type=jnp.float32,
        precision=jax.lax.Precision.HIGHEST).reshape(k, qn, 3)
    gl = jax.lax.dot_general(
        ohall, r_loa, dng, preferred_element_type=jnp.float32,
        precision=jax.lax.Precision.HIGHEST).reshape(k, qn, 3)
    gf = None
    if r_feats is not None:
        gf = jax.lax.dot_general(
            ohall, r_feats, dng, preferred_element_type=jnp.float32,
            precision=jax.lax.Precision.HIGHEST)
    rel = gx - q_xyz[None]
    dn = jnp.sqrt(jnp.sum(rel * rel, axis=-1, keepdims=True))
    u = rel / (dn + _EPS)
    c1 = jnp.sum(u * q_loa[None], axis=-1, keepdims=True)
    c2 = jnp.sum(u * gl, axis=-1, keepdims=True)
    c3 = jnp.sum(q_loa[None] * gl, axis=-1, keepdims=True)
    ri = jnp.concatenate([dn, c1, c2, c3], axis=-1).reshape(kq, 4)
    mm = (((1,), (0,)), ((), ()))
    h = jax.nn.relu(jax.lax.dot_general(
        ri, wri, mm, preferred_element_type=jnp.float32, precision=jax.lax.Precision.HIGHEST) + bri)
    z = jax.lax.dot_general(h, w0[0:64, :], mm,
                            preferred_element_type=jnp.float32, precision=jax.lax.Precision.HIGHEST)
    if r_feats is not None:
        z = z + jax.lax.dot_general(gf, w0[64:64 + cf, :], mm,
                                    preferred_element_type=jnp.float32, precision=jax.lax.Precision.HIGHEST)
    z = jax.nn.relu(z + b0)
    return jnp.max(z.reshape(k, qn, co), axis=0)


def _modules_body(xyz_ref, loa_ref,
                  nx1_ref, nl1_ref, nx2_ref, nl2_ref,
                  nx3_ref, nl3_ref, nx4_ref, nl4_ref,
                  am1_ref, am2_ref, am3_ref, am4_ref,
                  w1ri, b1ri, w10, b10, w2ri, b2ri, w20, b20,
                  w3ri, b3ri, w30, b30, w4ri, b4ri, w40, b40,
                  w5ri, b5ri, w50, b50,
                  out_ref):
    xyz = xyz_ref[0]
    loa = loa_ref[0]
    nx = [nx1_ref[0], nx2_ref[0], nx3_ref[0], nx4_ref[0]]
    nl = [nl1_ref[0], nl2_ref[0], nl3_ref[0], nl4_ref[0]]
    am = [am1_ref[0], am2_ref[0], am3_ref[0], am4_ref[0]]
    mp = [(w1ri, b1ri, w10, b10), (w2ri, b2ri, w20, b20),
          (w3ri, b3ri, w30, b30), (w4ri, b4ri, w40, b40)]

    f = None
    r_xyz, r_loa = xyz, loa
    for m in range(4):
        wri, bri, w0, b0 = mp[m]
        f = _run_module(nx[m], nl[m], r_xyz, r_loa, am[m], f,
                        _NSAMPLES[m],
                        wri[...], bri[...], w0[...], b0[...])
        r_xyz, r_loa = nx[m], nl[m]

    r_xyz, r_loa, r_feats = nx[3], nl[3], f
    q_xyz = jnp.mean(r_xyz, axis=0, keepdims=True)
    v5 = jnp.sum(r_loa, axis=0, keepdims=True)
    q_loa = v5 / (jnp.sqrt(jnp.sum(v5 * v5, axis=-1, keepdims=True)) + _EPS)
    rel = r_xyz - q_xyz
    dn = jnp.sqrt(jnp.sum(rel * rel, axis=-1, keepdims=True))
    u = rel / (dn + _EPS)
    c1 = jnp.sum(u * q_loa, axis=-1, keepdims=True)
    c2 = jnp.sum(u * r_loa, axis=-1, keepdims=True)
    c3 = jnp.sum(q_loa * r_loa, axis=-1, keepdims=True)
    ri = jnp.concatenate([dn, c1, c2, c3], axis=-1)
    mm = (((1,), (0,)), ((), ()))
    h = jax.nn.relu(jax.lax.dot_general(
        ri, w5ri[...], mm, preferred_element_type=jnp.float32, precision=jax.lax.Precision.HIGHEST) + b5ri[...])
    z = (jax.lax.dot_general(h, w50[0:64, :], mm,
                             preferred_element_type=jnp.float32, precision=jax.lax.Precision.HIGHEST)
         + jax.lax.dot_general(r_feats, w50[64:320, :], mm,
                               preferred_element_type=jnp.float32, precision=jax.lax.Precision.HIGHEST))
    z = jax.nn.relu(z + b50[...])
    out_ref[0] = jnp.max(z, axis=0, keepdims=True)


def _head_body(f5_ref, w1, b1, g1, bb1, w2, b2, g2, bb2, w3, b3, out_ref):
    mm = (((1,), (0,)), ((), ()))
    x = f5_ref[...]
    x = jax.nn.relu(g1[...] * (jax.lax.dot_general(
        x, w1[...], mm, preferred_element_type=jnp.float32, precision=jax.lax.Precision.HIGHEST) + b1[...])
        + bb1[...])
    x = jax.nn.relu(g2[...] * (jax.lax.dot_general(
        x, w2[...], mm, preferred_element_type=jnp.float32, precision=jax.lax.Precision.HIGHEST) + b2[...])
        + bb2[...])
    x = jax.lax.dot_general(
        x, w3[...], mm, preferred_element_type=jnp.float32, precision=jax.lax.Precision.HIGHEST) + b3[...]
    m = jnp.max(x, axis=-1, keepdims=True)
    lse = jnp.log(jnp.sum(jnp.exp(x - m), axis=-1, keepdims=True))
    out_ref[...] = x - m - lse


def _full_spec(shape):
    nd = len(shape)
    return pl.BlockSpec(shape, lambda *_a, _n=nd: (0,) * _n)


def kernel(xyz, params):
    b, n, _ = xyz.shape
    f32 = jnp.float32
    xyzT = jnp.transpose(xyz, (2, 0, 1))
    xyzP = jnp.transpose(xyz, (0, 2, 1))

    loa = pl.pallas_call(
        _loa_body,
        grid=(b,),
        in_specs=[pl.BlockSpec((1, n, 3), lambda i: (i, 0, 0)),
                  pl.BlockSpec((1, 3, n), lambda i: (i, 0, 0))],
        out_specs=pl.BlockSpec((1, n, 3), lambda i: (i, 0, 0)),
        out_shape=jax.ShapeDtypeStruct((b, n, 3), f32),
        scratch_shapes=[pltpu.VMEM((n, n), f32)],
        compiler_params=pltpu.CompilerParams(
            dimension_semantics=("parallel",)),
    )(xyz, xyzP)

    loaT = jnp.transpose(loa, (2, 0, 1))

    fps_outs = _sc_fps(xyzT, loaT, b)

    nx = [jnp.transpose(o[:, 0:3, :], (0, 2, 1)) for o in fps_outs]
    nl = [jnp.transpose(o[:, 3:6, :], (0, 2, 1)) for o in fps_outs]

    p = params
    mparams = []
    for m in range(1, 6):
        mparams += [p['m%d_Wri' % m], p['m%d_bri' % m].reshape(1, -1),
                    p['m%d_W0' % m], p['m%d_b0' % m].reshape(1, -1)]

    knn_cfg = [(2, 256, 1024, 8, xyzP, nx[0]),
               (2, 128, 256, 16, fps_outs[0], nx[1]),
               (2, 64, 128, 32, fps_outs[1], nx[2]),
               (2, 32, 64, 32, fps_outs[2], nx[3])]
    am_list = []
    for chunk, qn, nn, kk, rows_arr, q_arr in knn_cfg:
        am = pl.pallas_call(
            _make_knn_body(chunk, qn, nn, kk),
            grid=(b // chunk,),
            in_specs=[pl.BlockSpec((chunk, qn, 3), lambda i: (i, 0, 0)),
                      pl.BlockSpec((chunk,) + rows_arr.shape[1:],
                                   lambda i: (i, 0, 0))],
            out_specs=pl.BlockSpec((chunk, kk * qn, 1), lambda i: (i, 0, 0)),
            out_shape=jax.ShapeDtypeStruct((b, kk * qn, 1), jnp.int32),
            compiler_params=pltpu.CompilerParams(
                dimension_semantics=("parallel",)),
        )(q_arr, rows_arr)
        am_list.append(am)

    in_specs = [pl.BlockSpec((1, n, 3), lambda i: (i, 0, 0)),
                pl.BlockSpec((1, n, 3), lambda i: (i, 0, 0))]
    for pts in _NPOINTS:
        in_specs += [pl.BlockSpec((1, pts, 3), lambda i: (i, 0, 0))] * 2
    for pts, kk in zip(_NPOINTS, _NSAMPLES):
        in_specs.append(pl.BlockSpec((1, kk * pts, 1), lambda i: (i, 0, 0)))
    for w in mparams:
        in_specs.append(_full_spec(w.shape))

    args = [xyz, loa]
    for m in range(4):
        args += [nx[m], nl[m]]
    args += am_list
    args += mparams

    f5 = pl.pallas_call(
        _modules_body,
        grid=(b,),
        in_specs=in_specs,
        out_specs=pl.BlockSpec((1, 1, 512), lambda i: (i, 0, 0)),
        out_shape=jax.ShapeDtypeStruct((b, 1, 512), f32),
        compiler_params=pltpu.CompilerParams(
            dimension_semantics=("parallel",)),
    )(*args)

    hp = [p['fc1_W'], p['fc1_b'].reshape(1, -1),
          p['bn1_g'].reshape(1, -1), p['bn1_b'].reshape(1, -1),
          p['fc2_W'], p['fc2_b'].reshape(1, -1),
          p['bn2_g'].reshape(1, -1), p['bn2_b'].reshape(1, -1),
          p['fc3_W'], p['fc3_b'].reshape(1, -1)]
    logp = pl.pallas_call(
        _head_body,
        in_specs=[_full_spec((b, 512))] + [_full_spec(w.shape) for w in hp],
        out_specs=_full_spec((b, 40)),
        out_shape=jax.ShapeDtypeStruct((b, 40), f32),
    )(f5.reshape(b, 512), *hp)

    return logp, f5


# speedup vs baseline: 1.6159x; 1.2821x over previous
import functools

import jax
import jax.numpy as jnp
from jax import lax
from jax.experimental import pallas as pl
from jax.experimental.pallas import tpu as pltpu
from jax.experimental.pallas import tpu_sc as plsc

_NPOINTS = [256, 128, 64, 32]
_NSAMPLES = [8, 16, 32, 32]
_EPS = 1e-8
_BIG = 3.0e38


def _first_argmin_cols(x, iota, n):
    m = jnp.min(x, axis=1, keepdims=True)
    am = jnp.min(jnp.where(x == m, iota, n), axis=1, keepdims=True)
    return m, am


def _mimic_dists(q_xyz, r_rows):
    sq = jnp.sum(q_xyz * q_xyz, axis=1, keepdims=True)
    sr = (r_rows[0:1, :] * r_rows[0:1, :]
          + r_rows[1:2, :] * r_rows[1:2, :]
          + r_rows[2:3, :] * r_rows[2:3, :])
    g = jax.lax.dot_general(
        q_xyz, r_rows[0:3, :],
        (((1,), (0,)), ((), ())), preferred_element_type=jnp.float32)
    return (sq + sr) - 2.0 * g


def _loa_body(xyz_ref, xyzP_ref, out_ref, d_s):
    x = xyz_ref[0]
    xp = xyzP_ref[0]
    n = x.shape[0]
    d2 = jnp.zeros((n, n), jnp.float32)
    for c in range(3):
        col = x[:, c:c + 1]
        row = xp[c:c + 1, :]
        diff = col - row
        d2 = d2 + diff * diff
    d_s[...] = jnp.sqrt(d2)
    md = _mimic_dists(x, xp)
    def step(_, dw):
        m = jnp.min(dw, axis=1, keepdims=True)
        return jnp.where(dw <= m, _BIG, dw)

    dw = jax.lax.fori_loop(0, 32, step, md)
    msk = (dw == _BIG).astype(jnp.float32)
    d = d_s[...]
    t = jnp.max(msk * d, axis=1, keepdims=True)
    w = msk * (t - d)
    v = (jax.lax.dot_general(w, x, (((1,), (0,)), ((), ())),
                             preferred_element_type=jnp.float32, precision=jax.lax.Precision.HIGHEST)
         - jnp.sum(w, axis=1, keepdims=True) * x)
    nrm = jnp.sqrt(jnp.sum(v * v, axis=1, keepdims=True))
    out_ref[0] = v / (nrm + _EPS)


def _sc_level(srcs, dsts, dist, n, npoint):
    nsl = n // 16
    lane = lax.broadcasted_iota(jnp.int32, (16,), 0)

    def init_s(s, carry):
        dist[pl.ds(s * 16, 16)] = jnp.full((16,), 1e10, jnp.float32)
        return carry

    lax.fori_loop(0, nsl, init_s, 0)

    def body(i, far):
        cs = [srcs[c][pl.ds(far, 16)][0] for c in range(6)]
        blk = (i // 16) * 16
        sel_st = lane == (i - blk)
        for c in range(6):
            old = dsts[c][pl.ds(blk, 16)]
            dsts[c][pl.ds(blk, 16)] = jnp.where(
                sel_st, lax.broadcast(cs[c], (16,)), old)
        csv = [lax.broadcast(v, (16,)) for v in cs[:3]]

        def scan_s(s, carry):
            mv, ms = carry
            off = s * 16
            dx = srcs[0][pl.ds(off, 16)] - csv[0]
            dy = srcs[1][pl.ds(off, 16)] - csv[1]
            dz = srcs[2][pl.ds(off, 16)] - csv[2]
            d = dx * dx + dy * dy + dz * dz
            dn = jnp.minimum(dist[pl.ds(off, 16)], d)
            dist[pl.ds(off, 16)] = dn
            upd = dn > mv
            mv = jnp.where(upd, dn, mv)
            ms = jnp.where(upd, lax.broadcast(s, (16,)), ms)
            return mv, ms

        mv, ms = lax.fori_loop(
            0, nsl, scan_s, (jnp.full((16,), -1.0, jnp.float32),
                             jnp.zeros((16,), jnp.int32)))
        best = jnp.array(-1.0, jnp.float32)
        bidx = jnp.array(0, jnp.int32)
        for j in range(16):
            v = mv[j]
            cnd = ms[j] * 16 + j
            better = (v > best) | ((v == best) & (cnd < bidx))
            best = jnp.where(better, v, best)
            bidx = jnp.where(better, cnd, bidx)
        return bidx

    lax.fori_loop(0, npoint, body, jnp.array(0, jnp.int32))


def _sc_fps(xyzT, loaT, b):
    f32 = jnp.float32
    mesh = plsc.VectorSubcoreMesh(core_axis_name="c", subcore_axis_name="s")
    out_type = [jax.ShapeDtypeStruct((b, 6, p), f32) for p in _NPOINTS]
    scratch = ([pltpu.VMEM((1024,), f32)] * 6
               + [pltpu.VMEM((256,), f32)] * 6
               + [pltpu.VMEM((128,), f32)] * 6
               + [pltpu.VMEM((64,), f32)] * 6
               + [pltpu.VMEM((32,), f32)] * 6
               + [pltpu.VMEM((1024,), f32), pltpu.VMEM((32,), f32)])

    @functools.partial(pl.kernel, mesh=mesh, out_type=out_type,
                       scratch_types=scratch)
    def k(xyzT_hbm, loaT_hbm, o1, o2, o3, o4, *bufs):
        pin = list(bufs[0:6])
        lv = [list(bufs[6:12]), list(bufs[12:18]),
              list(bufs[18:24]), list(bufs[24:30])]
        dist = bufs[30]
        wid = lax.axis_index("s") * 2 + lax.axis_index("c")

        def sample(j, carry):
            bb = wid * 2 + j
            for c in range(3):
                pltpu.sync_copy(xyzT_hbm.at[c, bb], pin[c])
                pltpu.sync_copy(loaT_hbm.at[c, bb], pin[c + 3])
            for srcs, dsts, o_hbm, n, p in ((pin, lv[0], o1, 1024, 256),
                                            (lv[0], lv[1], o2, 256, 128),
                                            (lv[1], lv[2], o3, 128, 64),
                                            (lv[2], lv[3], o4, 64, 32)):
                _sc_level(srcs, dsts, dist, n, p)
                for c in range(6):
                    pltpu.sync_copy(dsts[c], o_hbm.at[bb, c])
            return carry

        lax.fori_loop(0, 2, sample, 0)

    return k(xyzT, loaT)


def _fps_level(planes, o_ref, npoint):
    b, n = planes[0].shape
    iota_n = jax.lax.broadcasted_iota(jnp.int32, (b, n), 1)
    iota_p = jax.lax.broadcasted_iota(jnp.int32, (b, npoint), 1)

    def body(i, st):
        dist, far, sel = st
        oh = (iota_n == far).astype(jnp.float32)
        cs = [jnp.sum(oh * a, axis=1, keepdims=True) for a in planes]
        d = ((planes[0] - cs[0]) ** 2 + (planes[1] - cs[1]) ** 2
             + (planes[2] - cs[2]) ** 2)
        dist = jnp.minimum(dist, d)
        m = jnp.max(dist, axis=1, keepdims=True)
        far = jnp.min(jnp.where(dist == m, iota_n, n), axis=1, keepdims=True)
        sel = tuple(jnp.where(iota_p == i, c, s) for c, s in zip(cs, sel))
        return dist, far, sel

    dist0 = jnp.full((b, n), 1e10, jnp.float32)
    far0 = jnp.zeros((b, 1), jnp.int32)
    sel0 = tuple(jnp.zeros((b, npoint), jnp.float32) for _ in range(6))
    _, _, sel = jax.lax.fori_loop(0, npoint, body, (dist0, far0, sel0))
    for c in range(6):
        o_ref[:, c, :] = sel[c]
    return list(sel)


def _fps_body(xyzT_ref, loaT_ref, o1, o2, o3, o4):
    planes = [xyzT_ref[c] for c in range(3)] + [loaT_ref[c] for c in range(3)]
    for o_ref, npoint in ((o1, _NPOINTS[0]), (o2, _NPOINTS[1]),
                          (o3, _NPOINTS[2]), (o4, _NPOINTS[3])):
        planes = _fps_level(planes, o_ref, npoint)


def _make_knn_body(chunk, q, n, k):
    def body(q_ref, rows_ref, out_ref):
        d2s = [_mimic_dists(q_ref[s], rows_ref[s]) for s in range(chunk)]
        d2 = jnp.concatenate(d2s, axis=0) if chunk > 1 else d2s[0]
        iota = jax.lax.broadcasted_iota(jnp.int32, (chunk * q, n), 1)

        def kstep(kk, d2c):
            _, am = _first_argmin_cols(d2c, iota, n)
            for s in range(chunk):
                out_ref[s, pl.ds(kk * q, q), 0:1] = am[s * q:(s + 1) * q]
            return jnp.where(iota == am, _BIG, d2c)

        jax.lax.fori_loop(0, k, kstep, d2)

    return body


def _run_module(q_xyz, q_loa, r_xyz, r_loa, am_col, r_feats, k,
                wri, bri, w0, b0):
    qn = q_xyz.shape[0]
    n = r_xyz.shape[0]
    cf = 0 if r_feats is None else r_feats.shape[1]
    co = w0.shape[1]
    kq = k * qn
    iota_kq = jax.lax.broadcasted_iota(jnp.int32, (kq, n), 1)
    ohall = (iota_kq == am_col).astype(jnp.float32)
    dng = (((1,), (0,)), ((), ()))
    gx = jax.lax.dot_general(
        ohall, r_xyz, dng, preferred_element_type=jnp.float32,
        precision=jax.lax.Precision.HIGHEST).reshape(k, qn, 3)
    gl = jax.lax.dot_general(
        ohall, r_loa, dng, preferred_element_type=jnp.float32,
        precision=jax.lax.Precision.HIGHEST).reshape(k, qn, 3)
    gf = None
    if r_feats is not None:
        gf = jax.lax.dot_general(
            ohall, r_feats, dng, preferred_element_type=jnp.float32,
            precision=jax.lax.Precision.HIGHEST)
    rel = gx - q_xyz[None]
    dn = jnp.sqrt(jnp.sum(rel * rel, axis=-1, keepdims=True))
    u = rel / (dn + _EPS)
    c1 = jnp.sum(u * q_loa[None], axis=-1, keepdims=True)
    c2 = jnp.sum(u * gl, axis=-1, keepdims=True)
    c3 = jnp.sum(q_loa[None] * gl, axis=-1, keepdims=True)
    ri = jnp.concatenate([dn, c1, c2, c3], axis=-1).reshape(kq, 4)
    mm = (((1,), (0,)), ((), ()))
    h = jax.nn.relu(jax.lax.dot_general(
        ri, wri, mm, preferred_element_type=jnp.float32, precision=jax.lax.Precision.HIGHEST) + bri)
    z = jax.lax.dot_general(h, w0[0:64, :], mm,
                            preferred_element_type=jnp.float32, precision=jax.lax.Precision.HIGHEST)
    if r_feats is not None:
        z = z + jax.lax.dot_general(gf, w0[64:64 + cf, :], mm,
                                    preferred_element_type=jnp.float32, precision=jax.lax.Precision.HIGHEST)
    z = jax.nn.relu(z + b0)
    return jnp.max(z.reshape(k, qn, co), axis=0)


def _modules_body(xyz_ref, loa_ref,
                  nx1_ref, nl1_ref, nx2_ref, nl2_ref,
                  nx3_ref, nl3_ref, nx4_ref, nl4_ref,
                  am1_ref, am2_ref, am3_ref, am4_ref,
                  w1ri, b1ri, w10, b10, w2ri, b2ri, w20, b20,
                  w3ri, b3ri, w30, b30, w4ri, b4ri, w40, b40,
                  w5ri, b5ri, w50, b50,
                  out_ref):
    xyz = xyz_ref[0]
    loa = loa_ref[0]
    nx = [nx1_ref[0], nx2_ref[0], nx3_ref[0], nx4_ref[0]]
    nl = [nl1_ref[0], nl2_ref[0], nl3_ref[0], nl4_ref[0]]
    am = [am1_ref[0], am2_ref[0], am3_ref[0], am4_ref[0]]
    mp = [(w1ri, b1ri, w10, b10), (w2ri, b2ri, w20, b20),
          (w3ri, b3ri, w30, b30), (w4ri, b4ri, w40, b40)]

    f = None
    r_xyz, r_loa = xyz, loa
    for m in range(4):
        wri, bri, w0, b0 = mp[m]
        f = _run_module(nx[m], nl[m], r_xyz, r_loa, am[m], f,
                        _NSAMPLES[m],
                        wri[...], bri[...], w0[...], b0[...])
        r_xyz, r_loa = nx[m], nl[m]

    r_xyz, r_loa, r_feats = nx[3], nl[3], f
    q_xyz = jnp.mean(r_xyz, axis=0, keepdims=True)
    v5 = jnp.sum(r_loa, axis=0, keepdims=True)
    q_loa = v5 / (jnp.sqrt(jnp.sum(v5 * v5, axis=-1, keepdims=True)) + _EPS)
    rel = r_xyz - q_xyz
    dn = jnp.sqrt(jnp.sum(rel * rel, axis=-1, keepdims=True))
    u = rel / (dn + _EPS)
    c1 = jnp.sum(u * q_loa, axis=-1, keepdims=True)
    c2 = jnp.sum(u * r_loa, axis=-1, keepdims=True)
    c3 = jnp.sum(q_loa * r_loa, axis=-1, keepdims=True)
    ri = jnp.concatenate([dn, c1, c2, c3], axis=-1)
    mm = (((1,), (0,)), ((), ()))
    h = jax.nn.relu(jax.lax.dot_general(
        ri, w5ri[...], mm, preferred_element_type=jnp.float32, precision=jax.lax.Precision.HIGHEST) + b5ri[...])
    z = (jax.lax.dot_general(h, w50[0:64, :], mm,
                             preferred_element_type=jnp.float32, precision=jax.lax.Precision.HIGHEST)
         + jax.lax.dot_general(r_feats, w50[64:320, :], mm,
                               preferred_element_type=jnp.float32, precision=jax.lax.Precision.HIGHEST))
    z = jax.nn.relu(z + b50[...])
    out_ref[0] = jnp.max(z, axis=0, keepdims=True)


def _head_body(f5_ref, w1, b1, g1, bb1, w2, b2, g2, bb2, w3, b3, out_ref):
    mm = (((1,), (0,)), ((), ()))
    x = f5_ref[...]
    x = jax.nn.relu(g1[...] * (jax.lax.dot_general(
        x, w1[...], mm, preferred_element_type=jnp.float32, precision=jax.lax.Precision.HIGHEST) + b1[...])
        + bb1[...])
    x = jax.nn.relu(g2[...] * (jax.lax.dot_general(
        x, w2[...], mm, preferred_element_type=jnp.float32, precision=jax.lax.Precision.HIGHEST) + b2[...])
        + bb2[...])
    x = jax.lax.dot_general(
        x, w3[...], mm, preferred_element_type=jnp.float32, precision=jax.lax.Precision.HIGHEST) + b3[...]
    m = jnp.max(x, axis=-1, keepdims=True)
    lse = jnp.log(jnp.sum(jnp.exp(x - m), axis=-1, keepdims=True))
    out_ref[...] = x - m - lse


def _full_spec(shape):
    nd = len(shape)
    return pl.BlockSpec(shape, lambda *_a, _n=nd: (0,) * _n)


def kernel(xyz, params):
    b, n, _ = xyz.shape
    f32 = jnp.float32
    xyzT = jnp.transpose(xyz, (2, 0, 1))
    xyzP = jnp.transpose(xyz, (0, 2, 1))

    loa = pl.pallas_call(
        _loa_body,
        grid=(b,),
        in_specs=[pl.BlockSpec((1, n, 3), lambda i: (i, 0, 0)),
                  pl.BlockSpec((1, 3, n), lambda i: (i, 0, 0))],
        out_specs=pl.BlockSpec((1, n, 3), lambda i: (i, 0, 0)),
        out_shape=jax.ShapeDtypeStruct((b, n, 3), f32),
        scratch_shapes=[pltpu.VMEM((n, n), f32)],
        compiler_params=pltpu.CompilerParams(
            dimension_semantics=("parallel",)),
    )(xyz, xyzP)

    loaT = jnp.transpose(loa, (2, 0, 1))

    fps_outs = _sc_fps(xyzT, loaT, b)

    nx = [jnp.transpose(o[:, 0:3, :], (0, 2, 1)) for o in fps_outs]
    nl = [jnp.transpose(o[:, 3:6, :], (0, 2, 1)) for o in fps_outs]

    p = params
    mparams = []
    for m in range(1, 6):
        mparams += [p['m%d_Wri' % m], p['m%d_bri' % m].reshape(1, -1),
                    p['m%d_W0' % m], p['m%d_b0' % m].reshape(1, -1)]

    knn_cfg = [(2, 256, 1024, 8, xyzP, nx[0]),
               (2, 128, 256, 16, fps_outs[0], nx[1]),
               (2, 64, 128, 32, fps_outs[1], nx[2]),
               (2, 32, 64, 32, fps_outs[2], nx[3])]
    am_list = []
    for chunk, qn, nn, kk, rows_arr, q_arr in knn_cfg:
        am = pl.pallas_call(
            _make_knn_body(chunk, qn, nn, kk),
            grid=(b // chunk,),
            in_specs=[pl.BlockSpec((chunk, qn, 3), lambda i: (i, 0, 0)),
                      pl.BlockSpec((chunk,) + rows_arr.shape[1:],
                                   lambda i: (i, 0, 0))],
            out_specs=pl.BlockSpec((chunk, kk * qn, 1), lambda i: (i, 0, 0)),
            out_shape=jax.ShapeDtypeStruct((b, kk * qn, 1), jnp.int32),
            compiler_params=pltpu.CompilerParams(
                dimension_semantics=("parallel",)),
        )(q_arr, rows_arr)
        am_list.append(am)

    in_specs = [pl.BlockSpec((1, n, 3), lambda i: (i, 0, 0)),
                pl.BlockSpec((1, n, 3), lambda i: (i, 0, 0))]
    for pts in _NPOINTS:
        in_specs += [pl.BlockSpec((1, pts, 3), lambda i: (i, 0, 0))] * 2
    for pts, kk in zip(_NPOINTS, _NSAMPLES):
        in_specs.append(pl.BlockSpec((1, kk * pts, 1), lambda i: (i, 0, 0)))
    for w in mparams:
        in_specs.append(_full_spec(w.shape))

    args = [xyz, loa]
    for m in range(4):
        args += [nx[m], nl[m]]
    args += am_list
    args += mparams

    f5 = pl.pallas_call(
        _modules_body,
        grid=(b,),
        in_specs=in_specs,
        out_specs=pl.BlockSpec((1, 1, 512), lambda i: (i, 0, 0)),
        out_shape=jax.ShapeDtypeStruct((b, 1, 512), f32),
        compiler_params=pltpu.CompilerParams(
            dimension_semantics=("parallel",)),
    )(*args)

    hp = [p['fc1_W'], p['fc1_b'].reshape(1, -1),
          p['bn1_g'].reshape(1, -1), p['bn1_b'].reshape(1, -1),
          p['fc2_W'], p['fc2_b'].reshape(1, -1),
          p['bn2_g'].reshape(1, -1), p['bn2_b'].reshape(1, -1),
          p['fc3_W'], p['fc3_b'].reshape(1, -1)]
    logp = pl.pallas_call(
        _head_body,
        in_specs=[_full_spec((b, 512))] + [_full_spec(w.shape) for w in hp],
        out_specs=_full_spec((b, 40)),
        out_shape=jax.ShapeDtypeStruct((b, 40), f32),
    )(f5.reshape(b, 512), *hp)

    return logp, f5


# speedup vs baseline: 1.6160x; 1.0001x over previous
import functools

import jax
import jax.numpy as jnp
from jax import lax
from jax.experimental import pallas as pl
from jax.experimental.pallas import tpu as pltpu
from jax.experimental.pallas import tpu_sc as plsc

_NPOINTS = [256, 128, 64, 32]
_NSAMPLES = [8, 16, 32, 32]
_EPS = 1e-8
_BIG = 3.0e38


def _first_argmin_cols(x, iota, n):
    m = jnp.min(x, axis=1, keepdims=True)
    am = jnp.min(jnp.where(x == m, iota, n), axis=1, keepdims=True)
    return m, am


def _mimic_dists(q_xyz, r_rows):
    sq = jnp.sum(q_xyz * q_xyz, axis=1, keepdims=True)
    sr = (r_rows[0:1, :] * r_rows[0:1, :]
          + r_rows[1:2, :] * r_rows[1:2, :]
          + r_rows[2:3, :] * r_rows[2:3, :])
    g = jax.lax.dot_general(
        q_xyz, r_rows[0:3, :],
        (((1,), (0,)), ((), ())), preferred_element_type=jnp.float32)
    return (sq + sr) - 2.0 * g


def _loa_body(xyz_ref, xyzP_ref, out_ref, d_s):
    x = xyz_ref[0]
    xp = xyzP_ref[0]
    n = x.shape[0]
    d2 = jnp.zeros((n, n), jnp.float32)
    for c in range(3):
        col = x[:, c:c + 1]
        row = xp[c:c + 1, :]
        diff = col - row
        d2 = d2 + diff * diff
    d_s[...] = jnp.sqrt(d2)
    md = _mimic_dists(x, xp)
    def step(_, dw):
        m = jnp.min(dw, axis=1, keepdims=True)
        return jnp.where(dw <= m, _BIG, dw)

    dw = jax.lax.fori_loop(0, 32, step, md)
    msk = (dw == _BIG).astype(jnp.float32)
    d = d_s[...]
    t = jnp.max(msk * d, axis=1, keepdims=True)
    w = msk * (t - d)
    v = (jax.lax.dot_general(w, x, (((1,), (0,)), ((), ())),
                             preferred_element_type=jnp.float32, precision=jax.lax.Precision.HIGHEST)
         - jnp.sum(w, axis=1, keepdims=True) * x)
    nrm = jnp.sqrt(jnp.sum(v * v, axis=1, keepdims=True))
    out_ref[0] = v / (nrm + _EPS)


def _sc_level(srcs, dsts, dist, n, npoint):
    nsl = n // 16
    lane = lax.broadcasted_iota(jnp.int32, (16,), 0)

    def init_s(s, carry):
        dist[pl.ds(s * 16, 16)] = jnp.full((16,), 1e10, jnp.float32)
        return carry

    lax.fori_loop(0, nsl, init_s, 0)

    def body(i, far):
        cs = [srcs[c][pl.ds(far, 16)][0] for c in range(6)]
        blk = (i // 16) * 16
        sel_st = lane == (i - blk)
        for c in range(6):
            old = dsts[c][pl.ds(blk, 16)]
            dsts[c][pl.ds(blk, 16)] = jnp.where(
                sel_st, lax.broadcast(cs[c], (16,)), old)
        csv = [lax.broadcast(v, (16,)) for v in cs[:3]]

        def scan_s(s, carry):
            mv, ms = carry
            off = s * 16
            dx = srcs[0][pl.ds(off, 16)] - csv[0]
            dy = srcs[1][pl.ds(off, 16)] - csv[1]
            dz = srcs[2][pl.ds(off, 16)] - csv[2]
            d = dx * dx + dy * dy + dz * dz
            dn = jnp.minimum(dist[pl.ds(off, 16)], d)
            dist[pl.ds(off, 16)] = dn
            upd = dn > mv
            mv = jnp.where(upd, dn, mv)
            ms = jnp.where(upd, lax.broadcast(s, (16,)), ms)
            return mv, ms

        mv, ms = lax.fori_loop(
            0, nsl, scan_s, (jnp.full((16,), -1.0, jnp.float32),
                             jnp.zeros((16,), jnp.int32)))
        best = jnp.array(-1.0, jnp.float32)
        bidx = jnp.array(0, jnp.int32)
        for j in range(16):
            v = mv[j]
            cnd = ms[j] * 16 + j
            better = (v > best) | ((v == best) & (cnd < bidx))
            best = jnp.where(better, v, best)
            bidx = jnp.where(better, cnd, bidx)
        return bidx

    lax.fori_loop(0, npoint, body, jnp.array(0, jnp.int32))


def _sc_fps(xyzT, loaT, b):
    f32 = jnp.float32
    mesh = plsc.VectorSubcoreMesh(core_axis_name="c", subcore_axis_name="s")
    out_type = [jax.ShapeDtypeStruct((b, 6, p), f32) for p in _NPOINTS]
    scratch = ([pltpu.VMEM((1024,), f32)] * 6
               + [pltpu.VMEM((256,), f32)] * 6
               + [pltpu.VMEM((128,), f32)] * 6
               + [pltpu.VMEM((64,), f32)] * 6
               + [pltpu.VMEM((32,), f32)] * 6
               + [pltpu.VMEM((1024,), f32), pltpu.VMEM((32,), f32)])

    @functools.partial(pl.kernel, mesh=mesh, out_type=out_type,
                       scratch_types=scratch)
    def k(xyzT_hbm, loaT_hbm, o1, o2, o3, o4, *bufs):
        pin = list(bufs[0:6])
        lv = [list(bufs[6:12]), list(bufs[12:18]),
              list(bufs[18:24]), list(bufs[24:30])]
        dist = bufs[30]
        wid = lax.axis_index("s") * 2 + lax.axis_index("c")

        def sample(j, carry):
            bb = wid * 2 + j
            for c in range(3):
                pltpu.sync_copy(xyzT_hbm.at[c, bb], pin[c])
                pltpu.sync_copy(loaT_hbm.at[c, bb], pin[c + 3])
            for srcs, dsts, o_hbm, n, p in ((pin, lv[0], o1, 1024, 256),
                                            (lv[0], lv[1], o2, 256, 128),
                                            (lv[1], lv[2], o3, 128, 64),
                                            (lv[2], lv[3], o4, 64, 32)):
                _sc_level(srcs, dsts, dist, n, p)
                for c in range(6):
                    pltpu.sync_copy(dsts[c], o_hbm.at[bb, c])
            return carry

        lax.fori_loop(0, 2, sample, 0)

    return k(xyzT, loaT)


def _make_knn_body(chunk, q, n, k):
    def body(q_ref, rows_ref, out_ref):
        d2s = [_mimic_dists(q_ref[s], rows_ref[s]) for s in range(chunk)]
        d2 = jnp.concatenate(d2s, axis=0) if chunk > 1 else d2s[0]
        iota = jax.lax.broadcasted_iota(jnp.int32, (chunk * q, n), 1)

        def kstep(kk, d2c):
            _, am = _first_argmin_cols(d2c, iota, n)
            for s in range(chunk):
                out_ref[s, pl.ds(kk * q, q), 0:1] = am[s * q:(s + 1) * q]
            return jnp.where(iota == am, _BIG, d2c)

        jax.lax.fori_loop(0, k, kstep, d2)

    return body


def _run_module(q_xyz, q_loa, r_xyz, r_loa, am_col, r_feats, k,
                wri, bri, w0, b0):
    qn = q_xyz.shape[0]
    n = r_xyz.shape[0]
    cf = 0 if r_feats is None else r_feats.shape[1]
    co = w0.shape[1]
    kq = k * qn
    iota_kq = jax.lax.broadcasted_iota(jnp.int32, (kq, n), 1)
    ohall = (iota_kq == am_col).astype(jnp.float32)
    dng = (((1,), (0,)), ((), ()))
    gx = jax.lax.dot_general(
        ohall, r_xyz, dng, preferred_element_type=jnp.float32,
        precision=jax.lax.Precision.HIGHEST).reshape(k, qn, 3)
    gl = jax.lax.dot_general(
        ohall, r_loa, dng, preferred_element_type=jnp.float32,
        precision=jax.lax.Precision.HIGHEST).reshape(k, qn, 3)
    gf = None
    if r_feats is not None:
        gf = jax.lax.dot_general(
            ohall, r_feats, dng, preferred_element_type=jnp.float32,
            precision=jax.lax.Precision.HIGHEST)
    rel = gx - q_xyz[None]
    dn = jnp.sqrt(jnp.sum(rel * rel, axis=-1, keepdims=True))
    u = rel / (dn + _EPS)
    c1 = jnp.sum(u * q_loa[None], axis=-1, keepdims=True)
    c2 = jnp.sum(u * gl, axis=-1, keepdims=True)
    c3 = jnp.sum(q_loa[None] * gl, axis=-1, keepdims=True)
    ri = jnp.concatenate([dn, c1, c2, c3], axis=-1).reshape(kq, 4)
    mm = (((1,), (0,)), ((), ()))
    h = jax.nn.relu(jax.lax.dot_general(
        ri, wri, mm, preferred_element_type=jnp.float32, precision=jax.lax.Precision.HIGHEST) + bri)
    z = jax.lax.dot_general(h, w0[0:64, :], mm,
                            preferred_element_type=jnp.float32, precision=jax.lax.Precision.HIGHEST)
    if r_feats is not None:
        z = z + jax.lax.dot_general(gf, w0[64:64 + cf, :], mm,
                                    preferred_element_type=jnp.float32, precision=jax.lax.Precision.HIGHEST)
    z = jax.nn.relu(z + b0)
    return jnp.max(z.reshape(k, qn, co), axis=0)


def _modules_body(xyz_ref, loa_ref,
                  nx1_ref, nl1_ref, nx2_ref, nl2_ref,
                  nx3_ref, nl3_ref, nx4_ref, nl4_ref,
                  am1_ref, am2_ref, am3_ref, am4_ref,
                  w1ri, b1ri, w10, b10, w2ri, b2ri, w20, b20,
                  w3ri, b3ri, w30, b30, w4ri, b4ri, w40, b40,
                  w5ri, b5ri, w50, b50,
                  out_ref):
    xyz = xyz_ref[0]
    loa = loa_ref[0]
    nx = [nx1_ref[0], nx2_ref[0], nx3_ref[0], nx4_ref[0]]
    nl = [nl1_ref[0], nl2_ref[0], nl3_ref[0], nl4_ref[0]]
    am = [am1_ref[0], am2_ref[0], am3_ref[0], am4_ref[0]]
    mp = [(w1ri, b1ri, w10, b10), (w2ri, b2ri, w20, b20),
          (w3ri, b3ri, w30, b30), (w4ri, b4ri, w40, b40)]

    f = None
    r_xyz, r_loa = xyz, loa
    for m in range(4):
        wri, bri, w0, b0 = mp[m]
        f = _run_module(nx[m], nl[m], r_xyz, r_loa, am[m], f,
                        _NSAMPLES[m],
                        wri[...], bri[...], w0[...], b0[...])
        r_xyz, r_loa = nx[m], nl[m]

    r_xyz, r_loa, r_feats = nx[3], nl[3], f
    q_xyz = jnp.mean(r_xyz, axis=0, keepdims=True)
    v5 = jnp.sum(r_loa, axis=0, keepdims=True)
    q_loa = v5 / (jnp.sqrt(jnp.sum(v5 * v5, axis=-1, keepdims=True)) + _EPS)
    rel = r_xyz - q_xyz
    dn = jnp.sqrt(jnp.sum(rel * rel, axis=-1, keepdims=True))
    u = rel / (dn + _EPS)
    c1 = jnp.sum(u * q_loa, axis=-1, keepdims=True)
    c2 = jnp.sum(u * r_loa, axis=-1, keepdims=True)
    c3 = jnp.sum(q_loa * r_loa, axis=-1, keepdims=True)
    ri = jnp.concatenate([dn, c1, c2, c3], axis=-1)
    mm = (((1,), (0,)), ((), ()))
    h = jax.nn.relu(jax.lax.dot_general(
        ri, w5ri[...], mm, preferred_element_type=jnp.float32, precision=jax.lax.Precision.HIGHEST) + b5ri[...])
    z = (jax.lax.dot_general(h, w50[0:64, :], mm,
                             preferred_element_type=jnp.float32, precision=jax.lax.Precision.HIGHEST)
         + jax.lax.dot_general(r_feats, w50[64:320, :], mm,
                               preferred_element_type=jnp.float32, precision=jax.lax.Precision.HIGHEST))
    z = jax.nn.relu(z + b50[...])
    out_ref[0] = jnp.max(z, axis=0, keepdims=True)


def _head_body(f5_ref, w1, b1, g1, bb1, w2, b2, g2, bb2, w3, b3, out_ref):
    mm = (((1,), (0,)), ((), ()))
    x = f5_ref[...]
    x = jax.nn.relu(g1[...] * (jax.lax.dot_general(
        x, w1[...], mm, preferred_element_type=jnp.float32, precision=jax.lax.Precision.HIGHEST) + b1[...])
        + bb1[...])
    x = jax.nn.relu(g2[...] * (jax.lax.dot_general(
        x, w2[...], mm, preferred_element_type=jnp.float32, precision=jax.lax.Precision.HIGHEST) + b2[...])
        + bb2[...])
    x = jax.lax.dot_general(
        x, w3[...], mm, preferred_element_type=jnp.float32, precision=jax.lax.Precision.HIGHEST) + b3[...]
    m = jnp.max(x, axis=-1, keepdims=True)
    lse = jnp.log(jnp.sum(jnp.exp(x - m), axis=-1, keepdims=True))
    out_ref[...] = x - m - lse


def _full_spec(shape):
    nd = len(shape)
    return pl.BlockSpec(shape, lambda *_a, _n=nd: (0,) * _n)


def kernel(xyz, params):
    b, n, _ = xyz.shape
    f32 = jnp.float32
    xyzT = jnp.transpose(xyz, (2, 0, 1))
    xyzP = jnp.transpose(xyz, (0, 2, 1))

    loa = pl.pallas_call(
        _loa_body,
        grid=(b,),
        in_specs=[pl.BlockSpec((1, n, 3), lambda i: (i, 0, 0)),
                  pl.BlockSpec((1, 3, n), lambda i: (i, 0, 0))],
        out_specs=pl.BlockSpec((1, n, 3), lambda i: (i, 0, 0)),
        out_shape=jax.ShapeDtypeStruct((b, n, 3), f32),
        scratch_shapes=[pltpu.VMEM((n, n), f32)],
        compiler_params=pltpu.CompilerParams(
            dimension_semantics=("parallel",)),
    )(xyz, xyzP)

    loaT = jnp.transpose(loa, (2, 0, 1))

    fps_outs = _sc_fps(xyzT, loaT, b)

    nx = [jnp.transpose(o[:, 0:3, :], (0, 2, 1)) for o in fps_outs]
    nl = [jnp.transpose(o[:, 3:6, :], (0, 2, 1)) for o in fps_outs]

    p = params
    mparams = []
    for m in range(1, 6):
        mparams += [p['m%d_Wri' % m], p['m%d_bri' % m].reshape(1, -1),
                    p['m%d_W0' % m], p['m%d_b0' % m].reshape(1, -1)]

    knn_cfg = [(2, 256, 1024, 8, xyzP, nx[0]),
               (2, 128, 256, 16, fps_outs[0], nx[1]),
               (2, 64, 128, 32, fps_outs[1], nx[2]),
               (2, 32, 64, 32, fps_outs[2], nx[3])]
    am_list = []
    for chunk, qn, nn, kk, rows_arr, q_arr in knn_cfg:
        am = pl.pallas_call(
            _make_knn_body(chunk, qn, nn, kk),
            grid=(b // chunk,),
            in_specs=[pl.BlockSpec((chunk, qn, 3), lambda i: (i, 0, 0)),
                      pl.BlockSpec((chunk,) + rows_arr.shape[1:],
                                   lambda i: (i, 0, 0))],
            out_specs=pl.BlockSpec((chunk, kk * qn, 1), lambda i: (i, 0, 0)),
            out_shape=jax.ShapeDtypeStruct((b, kk * qn, 1), jnp.int32),
            compiler_params=pltpu.CompilerParams(
                dimension_semantics=("parallel",)),
        )(q_arr, rows_arr)
        am_list.append(am)

    in_specs = [pl.BlockSpec((1, n, 3), lambda i: (i, 0, 0)),
                pl.BlockSpec((1, n, 3), lambda i: (i, 0, 0))]
    for pts in _NPOINTS:
        in_specs += [pl.BlockSpec((1, pts, 3), lambda i: (i, 0, 0))] * 2
    for pts, kk in zip(_NPOINTS, _NSAMPLES):
        in_specs.append(pl.BlockSpec((1, kk * pts, 1), lambda i: (i, 0, 0)))
    for w in mparams:
        in_specs.append(_full_spec(w.shape))

    args = [xyz, loa]
    for m in range(4):
        args += [nx[m], nl[m]]
    args += am_list
    args += mparams

    f5 = pl.pallas_call(
        _modules_body,
        grid=(b,),
        in_specs=in_specs,
        out_specs=pl.BlockSpec((1, 1, 512), lambda i: (i, 0, 0)),
        out_shape=jax.ShapeDtypeStruct((b, 1, 512), f32),
        compiler_params=pltpu.CompilerParams(
            dimension_semantics=("parallel",)),
    )(*args)

    hp = [p['fc1_W'], p['fc1_b'].reshape(1, -1),
          p['bn1_g'].reshape(1, -1), p['bn1_b'].reshape(1, -1),
          p['fc2_W'], p['fc2_b'].reshape(1, -1),
          p['bn2_g'].reshape(1, -1), p['bn2_b'].reshape(1, -1),
          p['fc3_W'], p['fc3_b'].reshape(1, -1)]
    logp = pl.pallas_call(
        _head_body,
        in_specs=[_full_spec((b, 512))] + [_full_spec(w.shape) for w in hp],
        out_specs=_full_spec((b, 40)),
        out_shape=jax.ShapeDtypeStruct((b, 40), f32),
    )(f5.reshape(b, 512), *hp)

    return logp, f5
